# baseline (speedup 1.0000x reference)
import jax
import jax.numpy as jnp
from jax import lax
from jax.experimental import pallas as pl
from jax.experimental.pallas import tpu as pltpu
from jax.experimental.pallas import tpu_sc as plsc

_N = 10000
_NP = 10240
_E = 160000
_K = 80
_NSUB = 16
_RPT = _NP // _NSUB
_CPT = 128
_EP = _CPT * _NSUB * _K


def _proj1_body(x_ref, w_ref, as_ref, ad_ref, xh0, xh1, aso, ado):
    xl = jnp.dot(x_ref[...], w_ref[...], preferred_element_type=jnp.float32)
    xh0[...] = xl[:, 0:256].astype(jnp.bfloat16)
    xh1[...] = xl[:, 256:512].astype(jnp.bfloat16)
    aso[...] = jnp.dot(xl, as_ref[...], preferred_element_type=jnp.float32)
    ado[...] = jnp.dot(xl, ad_ref[...], preferred_element_type=jnp.float32)


def _proj1(x, W1, As1, Ad1):
    n, cin = x.shape
    hid = W1.shape[1]
    BN = 640
    return pl.pallas_call(
        _proj1_body,
        grid=(n // BN,),
        in_specs=[
            pl.BlockSpec((BN, cin), lambda i: (i, 0)),
            pl.BlockSpec((cin, hid), lambda i: (0, 0)),
            pl.BlockSpec((hid, 16), lambda i: (0, 0)),
            pl.BlockSpec((hid, 16), lambda i: (0, 0)),
        ],
        out_specs=[pl.BlockSpec((BN, 256), lambda i: (i, 0))] * 2
        + [pl.BlockSpec((BN, 16), lambda i: (i, 0))] * 2,
        out_shape=[jax.ShapeDtypeStruct((n, 256), jnp.bfloat16)] * 2
        + [jax.ShapeDtypeStruct((n, 16), jnp.float32)] * 2,
    )(x, W1, As1, Ad1)


def _proj2_body(a0, a1, s0_ref, s1_ref, b_ref, w_ref, as_ref, ad_ref,
                xg0, xg1, aso, ado):
    s = s0_ref[...] + s1_ref[...]
    cols = []
    for hd in range(8):
        src_arr = a0 if hd < 4 else a1
        blk = src_arr[:, 64 * (hd % 4):64 * (hd % 4) + 64].astype(jnp.float32)
        d = s[:, hd:hd + 1] + 1e-16
        cols.append(blk / d)
    out1 = jnp.concatenate(cols, axis=1) + b_ref[...]
    h = jnp.where(out1 > 0, out1, jnp.exp(jnp.minimum(out1, 0.0)) - 1.0)
    xl2 = jnp.dot(h, w_ref[...], preferred_element_type=jnp.float32)
    xg0[...] = xl2[:, 0:128].astype(jnp.bfloat16)
    xg1[...] = xl2[:, 128:256].astype(jnp.bfloat16)
    aso[...] = jnp.dot(xl2, as_ref[...], preferred_element_type=jnp.float32)
    ado[...] = jnp.dot(xl2, ad_ref[...], preferred_element_type=jnp.float32)


def _proj2(acc, s1a, s1b, b1, W2, As2, Ad2):
    n = s1a.shape[0]
    hid = W2.shape[0]
    cout = W2.shape[1]
    BN = 640
    return pl.pallas_call(
        _proj2_body,
        grid=(n // BN,),
        in_specs=[pl.BlockSpec((BN, 256), lambda i: (i, 0))] * 2
        + [
            pl.BlockSpec((BN, 16), lambda i: (i, 0)),
            pl.BlockSpec((BN, 16), lambda i: (i, 0)),
            pl.BlockSpec((1, hid), lambda i: (0, 0)),
            pl.BlockSpec((hid, cout), lambda i: (0, 0)),
            pl.BlockSpec((cout, 16), lambda i: (0, 0)),
            pl.BlockSpec((cout, 16), lambda i: (0, 0)),
        ],
        out_specs=[pl.BlockSpec((BN, 128), lambda i: (i, 0))] * 2
        + [pl.BlockSpec((BN, 16), lambda i: (i, 0))] * 2,
        out_shape=[jax.ShapeDtypeStruct((n, 128), jnp.bfloat16)] * 2
        + [jax.ShapeDtypeStruct((n, 16), jnp.float32)] * 2,
    )(*acc, s1a, s1b, b1, W2, As2, Ad2)


def _final_body(a0, a1, s0_ref, s1_ref, b_ref, o_ref):
    d = s0_ref[:, 0:1] + s1_ref[:, 0:1] + 1e-16
    cat = jnp.concatenate([a0[...], a1[...]], axis=1).astype(jnp.float32)
    o_ref[...] = cat / d + b_ref[...]


def _final(acc, s2a, s2b, b2):
    n = s2a.shape[0]
    cout = b2.shape[1]
    BN = 640
    return pl.pallas_call(
        _final_body,
        grid=(n // BN,),
        in_specs=[pl.BlockSpec((BN, 128), lambda i: (i, 0))] * 2
        + [
            pl.BlockSpec((BN, 16), lambda i: (i, 0)),
            pl.BlockSpec((BN, 16), lambda i: (i, 0)),
            pl.BlockSpec((1, cout), lambda i: (0, 0)),
        ],
        out_specs=pl.BlockSpec((BN, cout), lambda i: (i, 0)),
        out_shape=jax.ShapeDtypeStruct((n, cout), jnp.float32),
    )(*acc, s2a, s2b, b2)


_GATHER_DNUMS = lax.GatherDimensionNumbers(
    offset_dims=(), collapsed_slice_dims=(0,), start_index_map=(0,))


def _lane_bcast(row16, h):
    idx = jnp.full((16, 1), h, jnp.int32)
    return lax.gather(row16, idx, _GATHER_DNUMS, slice_sizes=(1,),
                      mode=lax.GatherScatterMode.PROMISE_IN_BOUNDS)


def _make_edge_body(head_map_c0, head_map_c1):
    def body(src_h, dst_h, as_h, ad_h, tab0, tab1, zw, z16, o0, o1, s_o0, s_o1,
             acc_sh, s_sh, srcv0, srcv1, dstv0, dstv1, dsts0, dsts1,
             asv0, asv1, adv0, adv1, rows0, rows1,
             ib0, ib1, gs0, gs1, ss0, ss1):
        srcv = (srcv0, srcv1)
        dstv = (dstv0, dstv1)
        dsts = (dsts0, dsts1)
        asv = (asv0, asv1)
        adv = (adv0, adv1)
        pv = (asv0, asv1)
        rows = (rows0, rows1)
        ib = (ib0, ib1)
        gs = (gs0, gs1)
        ss = (ss0, ss1)

        core = lax.axis_index("c")
        tid = lax.axis_index("s")
        nslice = pl.ds(tid * _RPT, _RPT)

        def base_of(j):
            return (tid * _CPT + j) * _K

        def issue_idx(j, b):
            pltpu.async_copy(src_h.at[pl.ds(base_of(j), _K)], srcv[b], ib[b])
            pltpu.async_copy(dst_h.at[pl.ds(base_of(j), _K)], dstv[b], ib[b])

        def wait_idx(j, b):
            pltpu.make_async_copy(
                src_h.at[pl.ds(base_of(j), _K)], srcv[b], ib[b]).wait()
            pltpu.make_async_copy(
                dst_h.at[pl.ds(base_of(j), _K)], dstv[b], ib[b]).wait()

        def issue_gathers(j, b):
            pltpu.async_copy(dst_h.at[pl.ds(base_of(j), _K)], dsts[b], gs[b])
            pltpu.async_copy(as_h.at[srcv[b]], asv[b], gs[b])
            pltpu.async_copy(ad_h.at[dstv[b]], adv[b], gs[b])

            @pl.when(core == 0)
            def _():
                pltpu.async_copy(tab0.at[srcv[b]], rows[b], gs[b])

            @pl.when(core == 1)
            def _():
                pltpu.async_copy(tab1.at[srcv[b]], rows[b], gs[b])

        def wait_gathers(j, b):
            pltpu.make_async_copy(
                dst_h.at[pl.ds(base_of(j), _K)], dsts[b], gs[b]).wait()
            pltpu.make_async_copy(as_h.at[srcv[b]], asv[b], gs[b]).wait()
            pltpu.make_async_copy(ad_h.at[dstv[b]], adv[b], gs[b]).wait()
            pltpu.make_async_copy(tab0.at[srcv[b]], rows[b], gs[b]).wait()

        def scale_loop(b, head_map, store_p):
            heads = sorted(set(head_map))

            @plsc.parallel_loop(0, _K, unroll=8)
            def _(e):
                a = asv[b][e, :] + adv[b][e, :]
                prow = jnp.exp(jnp.maximum(a, a * 0.2))
                if store_p:
                    pv[b][e, :] = prow
                pb = {}
                for h in heads:
                    pf = _lane_bcast(prow, h)
                    pb[h] = plsc.pack(pf, pf,
                                      format=plsc.PackFormat.INTERLEAVED)
                for i, h in enumerate(head_map):
                    rows[b][e, pl.ds(32 * i, 32)] = (
                        rows[b][e, pl.ds(32 * i, 32)] * pb[h])

        def compute(b):
            if head_map_c0 == head_map_c1:
                scale_loop(b, head_map_c0, True)
            else:
                @pl.when(core == 0)
                def _():
                    scale_loop(b, head_map_c0, True)

                @pl.when(core == 1)
                def _():
                    scale_loop(b, head_map_c1, True)

        def issue_scatter(b):
            @pl.when(core == b)
            def _():
                pltpu.async_copy(pv[b], s_sh.at[dsts[b]], ss[b], add=True)
            pltpu.async_copy(rows[b], acc_sh.at[dsts[b]], ss[b], add=True)

        def wait_scatter(b):
            @pl.when(core == b)
            def _():
                pltpu.make_async_copy(pv[b], s_sh.at[dsts[b]], ss[b]).wait()
            pltpu.make_async_copy(rows[b], acc_sh.at[dsts[b]], ss[b]).wait()

        pltpu.sync_copy(zw, acc_sh.at[nslice])
        pltpu.sync_copy(z16, s_sh.at[nslice])
        plsc.subcore_barrier()

        issue_idx(0, 0)
        issue_idx(1, 1)
        wait_idx(0, 0)
        issue_gathers(0, 0)

        def steady(j, b):
            nb = 1 - b

            @pl.when(j + 1 < _CPT)
            def _():
                wait_idx(j + 1, nb)

            @pl.when(j >= 1)
            def _():
                wait_scatter(nb)

            @pl.when(j + 1 < _CPT)
            def _():
                issue_gathers(j + 1, nb)

            wait_gathers(j, b)

            @pl.when(j + 2 < _CPT)
            def _():
                issue_idx(j + 2, b)

            compute(b)
            issue_scatter(b)

        @pl.loop(0, _CPT // 2)
        def _(i):
            steady(2 * i, 0)
            steady(2 * i + 1, 1)

        wait_scatter((_CPT - 1) & 1)
        plsc.subcore_barrier()

        @pl.when(core == 0)
        def _():
            pltpu.sync_copy(acc_sh.at[nslice], o0.at[nslice])

        @pl.when(core == 1)
        def _():
            pltpu.sync_copy(acc_sh.at[nslice], o1.at[nslice])

        @pl.when(core == 0)
        def _():
            pltpu.sync_copy(s_sh.at[nslice], s_o0.at[nslice])

        @pl.when(core == 1)
        def _():
            pltpu.sync_copy(s_sh.at[nslice], s_o1.at[nslice])
        plsc.subcore_barrier()

    return body


def _edge_call(head_map_c0, head_map_c1,
               src, dst, as_tab, ad_tab, tabs, zw, z16):
    width = 32 * len(head_map_c0)
    mesh = plsc.VectorSubcoreMesh(core_axis_name="c", subcore_axis_name="s")
    f = pl.kernel(
        _make_edge_body(head_map_c0, head_map_c1),
        out_type=[jax.ShapeDtypeStruct((_NP, width), jnp.bfloat16)] * 2
        + [jax.ShapeDtypeStruct((_NP, 16), jnp.float32)] * 2,
        mesh=mesh,
        compiler_params=pltpu.CompilerParams(
            use_tc_tiling_on_sc=False, needs_layout_passes=False),
        scratch_types=[
            pltpu.VMEM_SHARED((_NP, width), jnp.bfloat16),
            pltpu.VMEM_SHARED((_NP, 16), jnp.float32),
            pltpu.VMEM((_K,), jnp.int32),
            pltpu.VMEM((_K,), jnp.int32),
            pltpu.VMEM((_K,), jnp.int32),
            pltpu.VMEM((_K,), jnp.int32),
            pltpu.VMEM((_K,), jnp.int32),
            pltpu.VMEM((_K,), jnp.int32),
            pltpu.VMEM((_K, 16), jnp.float32),
            pltpu.VMEM((_K, 16), jnp.float32),
            pltpu.VMEM((_K, 16), jnp.float32),
            pltpu.VMEM((_K, 16), jnp.float32),
            pltpu.VMEM((_K, width), jnp.bfloat16),
            pltpu.VMEM((_K, width), jnp.bfloat16),
            pltpu.SemaphoreType.DMA,
            pltpu.SemaphoreType.DMA,
            pltpu.SemaphoreType.DMA,
            pltpu.SemaphoreType.DMA,
            pltpu.SemaphoreType.DMA,
            pltpu.SemaphoreType.DMA,
        ],
    )
    return f(src, dst, as_tab, ad_tab, *tabs, zw, z16)


def _att_mats(att_src, att_dst):
    heads, head_dim = att_src.shape
    eye = jnp.eye(heads, dtype=att_src.dtype)
    a_s = (eye[:, None, :] * att_src[:, :, None]).reshape(heads * head_dim, heads)
    a_d = (eye[:, None, :] * att_dst[:, :, None]).reshape(heads * head_dim, heads)
    pad = 16 - heads
    return (jnp.pad(a_s, ((0, 0), (0, pad))), jnp.pad(a_d, ((0, 0), (0, pad))))


def kernel(x, edge_index, W1, att_src1, att_dst1, b1, W2, att_src2, att_dst2, b2):
    epad = jnp.full((_EP - _E,), _N, jnp.int32)
    src = jnp.concatenate([edge_index[0], epad])
    dst = jnp.concatenate([edge_index[1], epad])
    x_p = jnp.pad(x, ((0, _NP - _N), (0, 0)))
    z256 = jnp.zeros((_RPT, 256), jnp.bfloat16)
    z128 = jnp.zeros((_RPT, 128), jnp.bfloat16)
    z16 = jnp.zeros((_RPT, 16), jnp.float32)

    As1, Ad1 = _att_mats(att_src1, att_dst1)
    xh0, xh1, as1_tab, ad1_tab = _proj1(x_p, W1, As1, Ad1)
    a0, a1, s1a, s1b = _edge_call(
        (0, 0, 1, 1, 2, 2, 3, 3), (4, 4, 5, 5, 6, 6, 7, 7),
        src, dst, as1_tab, ad1_tab, (xh0, xh1), z256, z16)

    As2, Ad2 = _att_mats(att_src2, att_dst2)
    y0, y1, as2_tab, ad2_tab = _proj2((a0, a1), s1a, s1b,
                                      b1.reshape(1, -1), W2, As2, Ad2)
    c0, c1, s2a, s2b = _edge_call(
        (0, 0, 0, 0), (0, 0, 0, 0),
        src, dst, as2_tab, ad2_tab, (y0, y1), z128, z16)
    return _final((c0, c1), s2a, s2b, b2.reshape(1, -1))[:_N]


# speedup vs baseline: 1.0284x; 1.0284x over previous
import jax
import jax.numpy as jnp
from jax import lax
from jax.experimental import pallas as pl
from jax.experimental.pallas import tpu as pltpu
from jax.experimental.pallas import tpu_sc as plsc

_N = 10000
_NP = 10240
_E = 160000
_K = 64
_NSUB = 16
_RPT = _NP // _NSUB
_CPT = 160
_EP = _CPT * _NSUB * _K


def _proj1_body(x_ref, w_ref, as_ref, ad_ref, xh0, xh1, aso, ado):
    xl = jnp.dot(x_ref[...], w_ref[...], preferred_element_type=jnp.float32)
    xh0[...] = xl[:, 0:256].astype(jnp.bfloat16)
    xh1[...] = xl[:, 256:512].astype(jnp.bfloat16)
    aso[...] = jnp.dot(xl, as_ref[...], preferred_element_type=jnp.float32)
    ado[...] = jnp.dot(xl, ad_ref[...], preferred_element_type=jnp.float32)


def _proj1(x, W1, As1, Ad1):
    n, cin = x.shape
    hid = W1.shape[1]
    BN = 640
    return pl.pallas_call(
        _proj1_body,
        grid=(n // BN,),
        in_specs=[
            pl.BlockSpec((BN, cin), lambda i: (i, 0)),
            pl.BlockSpec((cin, hid), lambda i: (0, 0)),
            pl.BlockSpec((hid, 16), lambda i: (0, 0)),
            pl.BlockSpec((hid, 16), lambda i: (0, 0)),
        ],
        out_specs=[pl.BlockSpec((BN, 256), lambda i: (i, 0))] * 2
        + [pl.BlockSpec((BN, 16), lambda i: (i, 0))] * 2,
        out_shape=[jax.ShapeDtypeStruct((n, 256), jnp.bfloat16)] * 2
        + [jax.ShapeDtypeStruct((n, 16), jnp.float32)] * 2,
    )(x, W1, As1, Ad1)


def _proj2_body(a0, a1, s_ref, b_ref, w_ref, as_ref, ad_ref,
                xg0, xg1, aso, ado):
    cols = []
    for hd in range(8):
        src_arr = a0 if hd < 4 else a1
        blk = src_arr[:, 64 * (hd % 4):64 * (hd % 4) + 64].astype(jnp.float32)
        d = s_ref[:, hd:hd + 1] + 1e-16
        cols.append(blk / d)
    out1 = jnp.concatenate(cols, axis=1) + b_ref[...]
    h = jnp.where(out1 > 0, out1, jnp.exp(jnp.minimum(out1, 0.0)) - 1.0)
    xl2 = jnp.dot(h, w_ref[...], preferred_element_type=jnp.float32)
    xg0[...] = xl2[:, 0:128].astype(jnp.bfloat16)
    xg1[...] = xl2[:, 128:256].astype(jnp.bfloat16)
    aso[...] = jnp.dot(xl2, as_ref[...], preferred_element_type=jnp.float32)
    ado[...] = jnp.dot(xl2, ad_ref[...], preferred_element_type=jnp.float32)


def _proj2(acc, s1, b1, W2, As2, Ad2):
    n = s1.shape[0]
    hid = W2.shape[0]
    cout = W2.shape[1]
    BN = 640
    return pl.pallas_call(
        _proj2_body,
        grid=(n // BN,),
        in_specs=[pl.BlockSpec((BN, 256), lambda i: (i, 0))] * 2
        + [
            pl.BlockSpec((BN, 16), lambda i: (i, 0)),
            pl.BlockSpec((1, hid), lambda i: (0, 0)),
            pl.BlockSpec((hid, cout), lambda i: (0, 0)),
            pl.BlockSpec((cout, 16), lambda i: (0, 0)),
            pl.BlockSpec((cout, 16), lambda i: (0, 0)),
        ],
        out_specs=[pl.BlockSpec((BN, 128), lambda i: (i, 0))] * 2
        + [pl.BlockSpec((BN, 16), lambda i: (i, 0))] * 2,
        out_shape=[jax.ShapeDtypeStruct((n, 128), jnp.bfloat16)] * 2
        + [jax.ShapeDtypeStruct((n, 16), jnp.float32)] * 2,
    )(*acc, s1, b1, W2, As2, Ad2)


def _final_body(a0, a1, s_ref, b_ref, o_ref):
    d = s_ref[:, 0:1] + 1e-16
    cat = jnp.concatenate([a0[...], a1[...]], axis=1).astype(jnp.float32)
    o_ref[...] = cat / d + b_ref[...]


def _final(acc, s2, b2):
    n = s2.shape[0]
    cout = b2.shape[1]
    BN = 640
    return pl.pallas_call(
        _final_body,
        grid=(n // BN,),
        in_specs=[pl.BlockSpec((BN, 128), lambda i: (i, 0))] * 2
        + [
            pl.BlockSpec((BN, 16), lambda i: (i, 0)),
            pl.BlockSpec((1, cout), lambda i: (0, 0)),
        ],
        out_specs=pl.BlockSpec((BN, cout), lambda i: (i, 0)),
        out_shape=jax.ShapeDtypeStruct((n, cout), jnp.float32),
    )(*acc, s2, b2)


_GATHER_DNUMS = lax.GatherDimensionNumbers(
    offset_dims=(), collapsed_slice_dims=(0,), start_index_map=(0,))


def _lane_bcast(row16, h):
    idx = jnp.full((16, 1), h, jnp.int32)
    return lax.gather(row16, idx, _GATHER_DNUMS, slice_sizes=(1,),
                      mode=lax.GatherScatterMode.PROMISE_IN_BOUNDS)


def _make_edge_body(head_map_c0, head_map_c1, nbuf):
    def body(*refs):
        (src_h, dst_h, as_h, ad_h, tab0, tab1, zw, z16, o0, o1, s_o,
         acc_sh, s_sh) = refs[:13]
        rest = refs[13:]
        srcv = rest[0 * nbuf:1 * nbuf]
        dstv = rest[1 * nbuf:2 * nbuf]
        dsts = rest[2 * nbuf:3 * nbuf]
        asv = rest[3 * nbuf:4 * nbuf]
        adv = rest[4 * nbuf:5 * nbuf]
        pv = asv
        rows = rest[5 * nbuf:6 * nbuf]
        ib = rest[6 * nbuf:7 * nbuf]
        gs = rest[7 * nbuf:8 * nbuf]
        ss = rest[8 * nbuf:9 * nbuf]

        core = lax.axis_index("c")
        tid = lax.axis_index("s")
        nslice = pl.ds(tid * _RPT, _RPT)

        def base_of(j):
            return (tid * _CPT + j) * _K

        def issue_idx(j, b):
            pltpu.async_copy(src_h.at[pl.ds(base_of(j), _K)], srcv[b], ib[b])
            pltpu.async_copy(dst_h.at[pl.ds(base_of(j), _K)], dstv[b], ib[b])

        def wait_idx(j, b):
            pltpu.make_async_copy(
                src_h.at[pl.ds(base_of(j), _K)], srcv[b], ib[b]).wait()
            pltpu.make_async_copy(
                dst_h.at[pl.ds(base_of(j), _K)], dstv[b], ib[b]).wait()

        def issue_gathers(j, b):
            pltpu.async_copy(dst_h.at[pl.ds(base_of(j), _K)], dsts[b], gs[b])
            pltpu.async_copy(as_h.at[srcv[b]], asv[b], gs[b])
            pltpu.async_copy(ad_h.at[dstv[b]], adv[b], gs[b])

            @pl.when(core == 0)
            def _():
                pltpu.async_copy(tab0.at[srcv[b]], rows[b], gs[b])

            @pl.when(core == 1)
            def _():
                pltpu.async_copy(tab1.at[srcv[b]], rows[b], gs[b])

        def wait_gathers(j, b):
            pltpu.make_async_copy(
                dst_h.at[pl.ds(base_of(j), _K)], dsts[b], gs[b]).wait()
            pltpu.make_async_copy(as_h.at[srcv[b]], asv[b], gs[b]).wait()
            pltpu.make_async_copy(ad_h.at[dstv[b]], adv[b], gs[b]).wait()
            pltpu.make_async_copy(tab0.at[srcv[b]], rows[b], gs[b]).wait()

        def scale_loop(b, head_map, store_p):
            heads = sorted(set(head_map))

            @plsc.parallel_loop(0, _K, unroll=8)
            def _(e):
                a = asv[b][e, :] + adv[b][e, :]
                prow = jnp.exp(jnp.maximum(a, a * 0.2))
                if store_p:
                    pv[b][e, :] = prow
                pb = {}
                for h in heads:
                    pf = _lane_bcast(prow, h)
                    pb[h] = plsc.pack(pf, pf,
                                      format=plsc.PackFormat.INTERLEAVED)
                for i, h in enumerate(head_map):
                    rows[b][e, pl.ds(32 * i, 32)] = (
                        rows[b][e, pl.ds(32 * i, 32)] * pb[h])

        def compute(b):
            if head_map_c0 == head_map_c1:
                scale_loop(b, head_map_c0, True)
            else:
                @pl.when(core == 0)
                def _():
                    scale_loop(b, head_map_c0, True)

                @pl.when(core == 1)
                def _():
                    scale_loop(b, head_map_c1, False)

        def issue_scatter(b):
            @pl.when(core == 0)
            def _():
                pltpu.async_copy(pv[b], s_sh.at[dsts[b]], ss[b], add=True)
            pltpu.async_copy(rows[b], acc_sh.at[dsts[b]], ss[b], add=True)

        def wait_scatter(b):
            @pl.when(core == 0)
            def _():
                pltpu.make_async_copy(pv[b], s_sh.at[dsts[b]], ss[b]).wait()
            pltpu.make_async_copy(rows[b], acc_sh.at[dsts[b]], ss[b]).wait()

        pltpu.sync_copy(zw, acc_sh.at[nslice])

        @pl.when(core == 0)
        def _():
            pltpu.sync_copy(z16, s_sh.at[nslice])
        plsc.subcore_barrier()

        for t in range(nbuf):
            issue_idx(t, t)
        for t in range(nbuf - 1):
            wait_idx(t, t)
            issue_gathers(t, t)

        def steady(j, b):
            sl = (b + nbuf - 1) % nbuf

            @pl.when(j + nbuf - 1 < _CPT)
            def _():
                wait_idx(j + nbuf - 1, sl)

            @pl.when(j >= 1)
            def _():
                wait_scatter(sl)

            @pl.when(j + nbuf - 1 < _CPT)
            def _():
                issue_gathers(j + nbuf - 1, sl)

            wait_gathers(j, b)

            @pl.when(j + nbuf < _CPT)
            def _():
                issue_idx(j + nbuf, b)

            compute(b)
            issue_scatter(b)

        @pl.loop(0, _CPT // nbuf)
        def _(i):
            for t in range(nbuf):
                steady(nbuf * i + t, t)

        wait_scatter((_CPT - 1) % nbuf)
        plsc.subcore_barrier()

        @pl.when(core == 0)
        def _():
            pltpu.sync_copy(acc_sh.at[nslice], o0.at[nslice])

        @pl.when(core == 1)
        def _():
            pltpu.sync_copy(acc_sh.at[nslice], o1.at[nslice])

        @pl.when(core == 0)
        def _():
            pltpu.sync_copy(s_sh.at[nslice], s_o.at[nslice])
        plsc.subcore_barrier()

    return body


def _edge_call(head_map_c0, head_map_c1,
               src, dst, as_tab, ad_tab, tabs, zw, z16, nbuf=2):
    width = 32 * len(head_map_c0)
    mesh = plsc.VectorSubcoreMesh(core_axis_name="c", subcore_axis_name="s")
    f = pl.kernel(
        _make_edge_body(head_map_c0, head_map_c1, nbuf),
        out_type=[jax.ShapeDtypeStruct((_NP, width), jnp.bfloat16)] * 2
        + [jax.ShapeDtypeStruct((_NP, 16), jnp.float32)],
        mesh=mesh,
        compiler_params=pltpu.CompilerParams(
            use_tc_tiling_on_sc=False, needs_layout_passes=False),
        scratch_types=[
            pltpu.VMEM_SHARED((_NP, width), jnp.bfloat16),
            pltpu.VMEM_SHARED((_NP, 16), jnp.float32),
        ]
        + [pltpu.VMEM((_K,), jnp.int32)] * (3 * nbuf)
        + [pltpu.VMEM((_K, 16), jnp.float32)] * (2 * nbuf)
        + [pltpu.VMEM((_K, width), jnp.bfloat16)] * nbuf
        + [pltpu.SemaphoreType.DMA] * (3 * nbuf),
    )
    return f(src, dst, as_tab, ad_tab, *tabs, zw, z16)


def _att_mats(att_src, att_dst):
    heads, head_dim = att_src.shape
    eye = jnp.eye(heads, dtype=att_src.dtype)
    a_s = (eye[:, None, :] * att_src[:, :, None]).reshape(heads * head_dim, heads)
    a_d = (eye[:, None, :] * att_dst[:, :, None]).reshape(heads * head_dim, heads)
    pad = 16 - heads
    return (jnp.pad(a_s, ((0, 0), (0, pad))), jnp.pad(a_d, ((0, 0), (0, pad))))


def kernel(x, edge_index, W1, att_src1, att_dst1, b1, W2, att_src2, att_dst2, b2):
    epad = jnp.full((_EP - _E,), _N, jnp.int32)
    src = jnp.concatenate([edge_index[0], epad])
    dst = jnp.concatenate([edge_index[1], epad])
    x_p = jnp.pad(x, ((0, _NP - _N), (0, 0)))
    z256 = jnp.zeros((_RPT, 256), jnp.bfloat16)
    z128 = jnp.zeros((_RPT, 128), jnp.bfloat16)
    z16 = jnp.zeros((_RPT, 16), jnp.float32)

    As1, Ad1 = _att_mats(att_src1, att_dst1)
    xh0, xh1, as1_tab, ad1_tab = _proj1(x_p, W1, As1, Ad1)
    a0, a1, s1 = _edge_call(
        (0, 0, 1, 1, 2, 2, 3, 3), (4, 4, 5, 5, 6, 6, 7, 7),
        src, dst, as1_tab, ad1_tab, (xh0, xh1), z256, z16)

    As2, Ad2 = _att_mats(att_src2, att_dst2)
    y0, y1, as2_tab, ad2_tab = _proj2((a0, a1), s1,
                                      b1.reshape(1, -1), W2, As2, Ad2)
    c0, c1, s2 = _edge_call(
        (0, 0, 0, 0), (0, 0, 0, 0),
        src, dst, as2_tab, ad2_tab, (y0, y1), z128, z16, nbuf=4)
    return _final((c0, c1), s2, b2.reshape(1, -1))[:_N]
